# baseline (speedup 1.0000x reference)
import jax
import jax.numpy as jnp
from jax.experimental import pallas as pl

_EPS = 1e-8


def _simrel_kernel(x_ref, ca_ref, out_ref):
    ca = ca_ref[...]
    inv_ca = 1.0 / jnp.maximum(jnp.sqrt(jnp.sum(ca * ca, axis=1, keepdims=True)), _EPS)
    x = x_ref[0]
    inv_in = 1.0 / jnp.maximum(jnp.sqrt(jnp.sum(x * x, axis=1)), _EPS)
    dots = jax.lax.dot_general(
        ca, x,
        dimension_numbers=(((1,), (1,)), ((), ())),
        preferred_element_type=jnp.float32,
    )
    out_ref[0] = dots * inv_ca * inv_in[None, :]


def kernel(inputs, labels, class_avgs):
    del labels
    b, s, d = inputs.shape
    k = class_avgs.shape[0]
    out_t = pl.pallas_call(
        _simrel_kernel,
        grid=(b,),
        in_specs=[
            pl.BlockSpec((1, s, d), lambda i: (i, 0, 0)),
            pl.BlockSpec((k, d), lambda i: (0, 0)),
        ],
        out_specs=pl.BlockSpec((1, k, s), lambda i: (i, 0, 0)),
        out_shape=jax.ShapeDtypeStruct((b, k, s), jnp.float32),
    )(inputs, class_avgs)
    return jnp.swapaxes(out_t, 1, 2)
